# baseline (speedup 1.0000x reference)
import jax
import jax.numpy as jnp
import numpy as np
from jax.experimental import pallas as pl
from jax.experimental.pallas import tpu as pltpu

_NUM_LEARNERS = 1000
_NUM_TOPICS = 128
_NUM_KC = 256
_NUM_TIMES = 10
_DIM = 5
_SIGMA_V = 0.1
_SIGMA2_R = 0.1

_LOG_CONST = np.float32(-np.log(_SIGMA2_R) - 0.5 * np.log(2.0 * np.pi))
_INV_SIGMA = np.float32(1.0 / _SIGMA2_R)


_LBLK = 200
_NLB = _NUM_LEARNERS // _LBLK


def _kl_kernel(nw_ref, v_ref, noise_ref, w_ref, b_ref,
               u_ref, r_ref, c_ref, out_ref, vm_scr):
    step = pl.program_id(0) + pl.program_id(1)

    @pl.when(step == 0)
    def _init():
        nw = nw_ref[...]
        ones_col = jnp.ones((1, _NUM_TOPICS), dtype=jnp.float32)
        colsum = jax.lax.dot_general(
            nw, ones_col, (((0,), (1,)), ((), ())),
            preferred_element_type=jnp.float32)
        vm_scr[...] = colsum * v_ref[...] + noise_ref[...]
        out_ref[0, 0] = jnp.float32(0.0)

    ue = u_ref[0, 0] * w_ref[0, 0] + b_ref[0, 0]
    for d in range(1, _DIM):
        ue = ue + u_ref[0, d] * w_ref[0, d]
    mu_pre = jax.lax.dot_general(
        ue, vm_scr[...], (((1,), (1,)), ((), ())),
        preferred_element_type=jnp.float32)
    mu = jax.nn.sigmoid(mu_pre)
    z = (r_ref[0] - mu) * _INV_SIGMA
    lp = _LOG_CONST - 0.5 * (z * z)
    masked = jnp.where(c_ref[0] == 1, lp, jnp.float32(0.0))
    out_ref[0, 0] += jnp.sum(masked)


def kernel(Q, V, R, C, neighbor_weights, U, W_emb, b_emb):
    del Q
    u_t = jnp.transpose(U, (0, 3, 1, 2))
    noise = (jax.random.normal(jax.random.key(42), (_NUM_KC,), dtype=jnp.float32)
             * jnp.float32(_SIGMA_V)).reshape(1, _NUM_KC)
    b2 = b_emb.reshape(1, 1)

    full = lambda shape: pl.BlockSpec(shape, lambda i, j: (0,) * len(shape))
    out = pl.pallas_call(
        _kl_kernel,
        grid=(_NUM_TIMES, _NLB),
        in_specs=[
            full((_NUM_TOPICS, _NUM_TOPICS)),
            full((_NUM_TOPICS, _NUM_KC)),
            full((1, _NUM_KC)),
            pl.BlockSpec(memory_space=pltpu.MemorySpace.SMEM),
            pl.BlockSpec(memory_space=pltpu.MemorySpace.SMEM),
            pl.BlockSpec((1, _DIM, _LBLK, _NUM_KC),
                         lambda i, j: (i, 0, j, 0)),
            pl.BlockSpec((1, _LBLK, _NUM_TOPICS),
                         lambda i, j: (i, j, 0)),
            pl.BlockSpec((1, _LBLK, _NUM_TOPICS),
                         lambda i, j: (i, j, 0)),
        ],
        out_specs=pl.BlockSpec(memory_space=pltpu.MemorySpace.SMEM),
        out_shape=jax.ShapeDtypeStruct((1, 1), jnp.float32),
        scratch_shapes=[
            pltpu.VMEM((_NUM_TOPICS, _NUM_KC), jnp.float32),
        ],
    )(neighbor_weights, V, noise, W_emb, b2, u_t, R, C)
    return out[0, 0]


# speedup vs baseline: 2.0042x; 2.0042x over previous
import jax
import jax.numpy as jnp
import numpy as np
from jax.experimental import pallas as pl
from jax.experimental.pallas import tpu as pltpu

_NUM_LEARNERS = 1000
_NUM_TOPICS = 128
_NUM_KC = 256
_NUM_TIMES = 10
_DIM = 5
_SIGMA_V = 0.1
_SIGMA2_R = 0.1

_LOG_CONST = np.float32(-np.log(_SIGMA2_R) - 0.5 * np.log(2.0 * np.pi))
_INV_SIGMA = np.float32(1.0 / _SIGMA2_R)


_TBLK = 2


def _kl_kernel(nw_ref, v_ref, noise_ref, w_ref, b_ref,
               u_ref, r_ref, c_ref, out_ref, vm_scr):
    step = pl.program_id(0)

    @pl.when(step == 0)
    def _init():
        nw = nw_ref[...]
        ones_col = jnp.ones((1, _NUM_TOPICS), dtype=jnp.float32)
        colsum = jax.lax.dot_general(
            nw, ones_col, (((0,), (1,)), ((), ())),
            preferred_element_type=jnp.float32)
        vm_scr[...] = colsum * v_ref[...] + noise_ref[...]
        out_ref[0, 0] = jnp.float32(0.0)

    total = jnp.float32(0.0)
    for tt in range(_TBLK):
        ue = u_ref[tt, 0] * w_ref[0, 0] + b_ref[0, 0]
        for d in range(1, _DIM):
            ue = ue + u_ref[tt, d] * w_ref[0, d]
        mu_pre = jax.lax.dot_general(
            ue, vm_scr[...], (((1,), (1,)), ((), ())),
            preferred_element_type=jnp.float32)
        mu = jax.nn.sigmoid(mu_pre)
        z = (r_ref[tt] - mu) * _INV_SIGMA
        lp = _LOG_CONST - 0.5 * (z * z)
        masked = jnp.where(c_ref[tt] == 1, lp, jnp.float32(0.0))
        total = total + jnp.sum(masked)
    out_ref[0, 0] += total


def kernel(Q, V, R, C, neighbor_weights, U, W_emb, b_emb):
    del Q
    u_t = jnp.transpose(U, (0, 3, 1, 2))
    noise = (jax.random.normal(jax.random.key(42), (_NUM_KC,), dtype=jnp.float32)
             * jnp.float32(_SIGMA_V)).reshape(1, _NUM_KC)
    b2 = b_emb.reshape(1, 1)

    full = lambda shape: pl.BlockSpec(shape, lambda i: (0,) * len(shape))
    out = pl.pallas_call(
        _kl_kernel,
        grid=(_NUM_TIMES // _TBLK,),
        in_specs=[
            full((_NUM_TOPICS, _NUM_TOPICS)),
            full((_NUM_TOPICS, _NUM_KC)),
            full((1, _NUM_KC)),
            pl.BlockSpec(memory_space=pltpu.MemorySpace.SMEM),
            pl.BlockSpec(memory_space=pltpu.MemorySpace.SMEM),
            pl.BlockSpec((_TBLK, _DIM, _NUM_LEARNERS, _NUM_KC),
                         lambda i: (i, 0, 0, 0)),
            pl.BlockSpec((_TBLK, _NUM_LEARNERS, _NUM_TOPICS),
                         lambda i: (i, 0, 0)),
            pl.BlockSpec((_TBLK, _NUM_LEARNERS, _NUM_TOPICS),
                         lambda i: (i, 0, 0)),
        ],
        out_specs=pl.BlockSpec(memory_space=pltpu.MemorySpace.SMEM),
        out_shape=jax.ShapeDtypeStruct((1, 1), jnp.float32),
        scratch_shapes=[
            pltpu.VMEM((_NUM_TOPICS, _NUM_KC), jnp.float32),
        ],
    )(neighbor_weights, V, noise, W_emb, b2, u_t, R, C)
    return out[0, 0]


# speedup vs baseline: 2.0200x; 1.0079x over previous
import jax
import jax.numpy as jnp
import numpy as np
from jax.experimental import pallas as pl
from jax.experimental.pallas import tpu as pltpu

_NUM_LEARNERS = 1000
_NUM_TOPICS = 128
_NUM_KC = 256
_NUM_TIMES = 10
_DIM = 5
_SIGMA_V = 0.1
_SIGMA2_R = 0.1

_LOG_CONST = np.float32(-np.log(_SIGMA2_R) - 0.5 * np.log(2.0 * np.pi))
_INV_SIGMA = np.float32(1.0 / _SIGMA2_R)


_TBLK = 1


def _kl_kernel(nw_ref, v_ref, noise_ref, w_ref, b_ref,
               u_ref, r_ref, c_ref, out_ref, vm_scr):
    step = pl.program_id(0)

    @pl.when(step == 0)
    def _init():
        nw = nw_ref[...]
        ones_col = jnp.ones((1, _NUM_TOPICS), dtype=jnp.float32)
        colsum = jax.lax.dot_general(
            nw, ones_col, (((0,), (1,)), ((), ())),
            preferred_element_type=jnp.float32)
        vm_scr[...] = colsum * v_ref[...] + noise_ref[...]
        out_ref[0, 0] = jnp.float32(0.0)

    total = jnp.float32(0.0)
    for tt in range(_TBLK):
        ue = u_ref[tt, 0] * w_ref[0, 0] + b_ref[0, 0]
        for d in range(1, _DIM):
            ue = ue + u_ref[tt, d] * w_ref[0, d]
        mu_pre = jax.lax.dot_general(
            ue, vm_scr[...], (((1,), (1,)), ((), ())),
            preferred_element_type=jnp.float32)
        mu = jax.nn.sigmoid(mu_pre)
        z = (r_ref[tt] - mu) * _INV_SIGMA
        lp = _LOG_CONST - 0.5 * (z * z)
        masked = jnp.where(c_ref[tt] == 1, lp, jnp.float32(0.0))
        total = total + jnp.sum(masked)
    out_ref[0, 0] += total


def kernel(Q, V, R, C, neighbor_weights, U, W_emb, b_emb):
    del Q
    u_t = jnp.transpose(U, (0, 3, 1, 2))
    noise = (jax.random.normal(jax.random.key(42), (_NUM_KC,), dtype=jnp.float32)
             * jnp.float32(_SIGMA_V)).reshape(1, _NUM_KC)
    b2 = b_emb.reshape(1, 1)

    full = lambda shape: pl.BlockSpec(shape, lambda i: (0,) * len(shape))
    out = pl.pallas_call(
        _kl_kernel,
        grid=(_NUM_TIMES // _TBLK,),
        in_specs=[
            full((_NUM_TOPICS, _NUM_TOPICS)),
            full((_NUM_TOPICS, _NUM_KC)),
            full((1, _NUM_KC)),
            pl.BlockSpec(memory_space=pltpu.MemorySpace.SMEM),
            pl.BlockSpec(memory_space=pltpu.MemorySpace.SMEM),
            pl.BlockSpec((_TBLK, _DIM, _NUM_LEARNERS, _NUM_KC),
                         lambda i: (i, 0, 0, 0)),
            pl.BlockSpec((_TBLK, _NUM_LEARNERS, _NUM_TOPICS),
                         lambda i: (i, 0, 0)),
            pl.BlockSpec((_TBLK, _NUM_LEARNERS, _NUM_TOPICS),
                         lambda i: (i, 0, 0)),
        ],
        out_specs=pl.BlockSpec(memory_space=pltpu.MemorySpace.SMEM),
        out_shape=jax.ShapeDtypeStruct((1, 1), jnp.float32),
        scratch_shapes=[
            pltpu.VMEM((_NUM_TOPICS, _NUM_KC), jnp.float32),
        ],
    )(neighbor_weights, V, noise, W_emb, b2, u_t, R, C)
    return out[0, 0]
